# Initial kernel scaffold; baseline (speedup 1.0000x reference)
import jax
import jax.numpy as jnp
from jax.experimental import pallas as pl


def kernel(xyz, features, W1, b1, W2, b2):
    raise NotImplementedError("write your pallas kernel here")


# speedup vs baseline: 10.8114x; 10.8114x over previous
import functools

import jax
import jax.numpy as jnp
from jax import lax
from jax.experimental import pallas as pl
from jax.experimental.pallas import tpu as pltpu
from jax.experimental.pallas import tpu_sc as plsc

B, N = 4, 8192
NPOINT = 1024
NSAMPLE = 32
H1, H2 = 32, 64

_HI = jax.lax.Precision.HIGHEST


def _fps_body(x_ref, y_ref, z_ref, idx_ref, cx_ref, cy_ref, cz_ref,
              dists_ref, far_ref):
    i = pl.program_id(0)

    @pl.when(i == 0)
    def _init():
        dists_ref[...] = jnp.full((B, N), 1e10, dtype=jnp.float32)
        far_ref[...] = jnp.zeros((B, 1), dtype=jnp.int32)

    far = far_ref[...]
    idx_ref[...] = far.reshape(1, B, 1)

    ji = lax.broadcasted_iota(jnp.int32, (B, N), 1)
    m = ji == far
    X = x_ref[...]
    Y = y_ref[...]
    Z = z_ref[...]
    cx = jnp.sum(jnp.where(m, X, 0.0), axis=1, keepdims=True)
    cy = jnp.sum(jnp.where(m, Y, 0.0), axis=1, keepdims=True)
    cz = jnp.sum(jnp.where(m, Z, 0.0), axis=1, keepdims=True)
    cx_ref[...] = cx.reshape(1, B, 1)
    cy_ref[...] = cy.reshape(1, B, 1)
    cz_ref[...] = cz.reshape(1, B, 1)

    dx = X - cx
    dy = Y - cy
    dz = Z - cz
    d = (dx * dx + dy * dy) + dz * dz
    nd = jnp.minimum(dists_ref[...], d)
    dists_ref[...] = nd
    mx = jnp.max(nd, axis=1, keepdims=True)
    first = jnp.min(jnp.where(nd == mx, ji, N), axis=1, keepdims=True)
    far_ref[...] = first.astype(jnp.int32)


def _fps_call(xp, yp, zp):
    plane = pl.BlockSpec((B, N), lambda i: (0, 0))
    out = pl.BlockSpec((1, B, 1), lambda i: (i, 0, 0))
    return pl.pallas_call(
        _fps_body,
        grid=(NPOINT,),
        in_specs=[plane, plane, plane],
        out_specs=[out, out, out, out],
        out_shape=[
            jax.ShapeDtypeStruct((NPOINT, B, 1), jnp.int32),
            jax.ShapeDtypeStruct((NPOINT, B, 1), jnp.float32),
            jax.ShapeDtypeStruct((NPOINT, B, 1), jnp.float32),
            jax.ShapeDtypeStruct((NPOINT, B, 1), jnp.float32),
        ],
        scratch_shapes=[
            pltpu.VMEM((B, N), jnp.float32),
            pltpu.VMEM((B, 1), jnp.int32),
        ],
        compiler_params=pltpu.CompilerParams(
            dimension_semantics=("arbitrary",)),
    )(xp, yp, zp)


_SBLK = 128


def _knn_body(x_ref, y_ref, z_ref, sx_ref, sy_ref, sz_ref, idx_ref):
    X = x_ref[0]
    Y = y_ref[0]
    Z = z_ref[0]
    sx = sx_ref[0]
    sy = sy_ref[0]
    sz = sz_ref[0]
    x2 = (X * X + Y * Y) + Z * Z
    s2 = (sx * sx + sy * sy) + sz * sz
    bf = lambda a: a.astype(jnp.bfloat16).astype(jnp.float32)
    cross = (bf(sx) * bf(X) + bf(sy) * bf(Y)) + bf(sz) * bf(Z)
    d = (s2 + x2) - 2.0 * cross
    ji = lax.broadcasted_iota(jnp.int32, (_SBLK, N), 1)
    for k in range(NSAMPLE):
        mn = jnp.min(d, axis=1, keepdims=True)
        first = jnp.min(jnp.where(d == mn, ji, N), axis=1, keepdims=True)
        idx_ref[0, :, k:k + 1] = first.astype(jnp.int32)
        d = jnp.where(ji == first, jnp.inf, d)


def _knn_call(xp, yp, zp, sxt, syt, szt):
    plane = pl.BlockSpec((1, 1, N), lambda b, s: (b, 0, 0))
    cen = pl.BlockSpec((1, _SBLK, 1), lambda b, s: (b, s, 0))
    p3 = lambda a: a.reshape(B, 1, N)
    c3 = lambda a: a.reshape(B, NPOINT, 1)
    return pl.pallas_call(
        _knn_body,
        grid=(B, NPOINT // _SBLK),
        in_specs=[plane, plane, plane, cen, cen, cen],
        out_specs=pl.BlockSpec((1, _SBLK, NSAMPLE), lambda b, s: (b, s, 0)),
        out_shape=jax.ShapeDtypeStruct((B, NPOINT, NSAMPLE), jnp.int32),
        compiler_params=pltpu.CompilerParams(
            dimension_semantics=("parallel", "arbitrary")),
    )(p3(xp), p3(yp), p3(zp), c3(sxt), c3(syt), c3(szt))


_PBLK = 1024


def _pq_body(xyz_ref, feat_ref, w1a_ref, w1b_ref, b1_ref, pq_ref):
    g = jnp.dot(xyz_ref[0], w1a_ref[...], precision=_HI,
                preferred_element_type=jnp.float32)
    g = g + jnp.dot(feat_ref[0], w1b_ref[...], precision=_HI,
                    preferred_element_type=jnp.float32)
    pq_ref[...] = (g + b1_ref[...]).reshape(1, _PBLK, H1)


def _pq_call(xyz, feat, w1a, w1b, b1r):
    return pl.pallas_call(
        _pq_body,
        grid=(B, N // _PBLK),
        in_specs=[
            pl.BlockSpec((1, _PBLK, 3), lambda b, s: (b, s, 0)),
            pl.BlockSpec((1, _PBLK, feat.shape[-1]), lambda b, s: (b, s, 0)),
            pl.BlockSpec(w1a.shape, lambda b, s: (0, 0)),
            pl.BlockSpec(w1b.shape, lambda b, s: (0, 0)),
            pl.BlockSpec((1, H1), lambda b, s: (0, 0)),
        ],
        out_specs=pl.BlockSpec((1, _PBLK, H1), lambda b, s: (b, s, 0)),
        out_shape=jax.ShapeDtypeStruct((B, N, H1), jnp.float32),
        compiler_params=pltpu.CompilerParams(
            dimension_semantics=("parallel", "parallel")),
    )(xyz, feat, w1a, w1b, b1r)


_GCH = 1024
_NW = 32


def _gather_body(table_ref, idx_ref, out_ref, idx_v, rows_v, sem):
    wid = lax.axis_index("s") * 2 + lax.axis_index("c")
    per_w = (B * NPOINT * NSAMPLE) // _NW
    for ch in range(per_w // _GCH):
        base = wid * per_w + ch * _GCH
        pltpu.sync_copy(idx_ref.at[pl.ds(base, _GCH)], idx_v)
        pltpu.async_copy(table_ref.at[idx_v], rows_v, sem).wait()
        pltpu.sync_copy(rows_v, out_ref.at[pl.ds(base, _GCH)])


def _gather_call(table, flat_idx):
    total = B * NPOINT * NSAMPLE
    mesh = plsc.VectorSubcoreMesh(core_axis_name="c", subcore_axis_name="s")
    return pl.kernel(
        _gather_body,
        out_type=jax.ShapeDtypeStruct((total, H1), jnp.float32),
        mesh=mesh,
        scratch_types=[
            pltpu.VMEM((_GCH,), jnp.int32),
            pltpu.VMEM((_GCH, H1), jnp.float32),
            pltpu.SemaphoreType.DMA,
        ],
        compiler_params=pltpu.CompilerParams(use_tc_tiling_on_sc=False),
    )(table, flat_idx)


_CBLK = 128


def _mlp_body(g_ref, nx_ref, w1a_ref, w2_ref, b2_ref, out_ref):
    r = jnp.dot(nx_ref[...], w1a_ref[...], precision=_HI,
                preferred_element_type=jnp.float32)
    g = g_ref[...]
    h1 = jnp.maximum(g - r[:, None, :], 0.0)
    z = jnp.dot(h1.reshape(_CBLK * NSAMPLE, H1), w2_ref[...], precision=_HI,
                preferred_element_type=jnp.float32)
    z = jnp.maximum(z + b2_ref[...], 0.0)
    zr = z.reshape(_CBLK, NSAMPLE, H2)
    acc = zr[:, 0, :].reshape(_CBLK, H2)
    for j in range(1, NSAMPLE):
        acc = jnp.maximum(acc, zr[:, j, :].reshape(_CBLK, H2))
    out_ref[...] = acc


def _mlp_call(g3, nxf, w1a, w2, b2r):
    s = NPOINT * B // _CBLK
    return pl.pallas_call(
        _mlp_body,
        grid=(s,),
        in_specs=[
            pl.BlockSpec((_CBLK, NSAMPLE, H1), lambda i: (i, 0, 0)),
            pl.BlockSpec((_CBLK, 3), lambda i: (i, 0)),
            pl.BlockSpec(w1a.shape, lambda i: (0, 0)),
            pl.BlockSpec(w2.shape, lambda i: (0, 0)),
            pl.BlockSpec((1, H2), lambda i: (0, 0)),
        ],
        out_specs=pl.BlockSpec((_CBLK, H2), lambda i: (i, 0)),
        out_shape=jax.ShapeDtypeStruct((B * NPOINT, H2), jnp.float32),
        compiler_params=pltpu.CompilerParams(
            dimension_semantics=("arbitrary",)),
    )(g3, nxf, w1a, w2, b2r)


@jax.jit
def kernel(xyz, features, W1, b1, W2, b2):
    xp = xyz[:, :, 0]
    yp = xyz[:, :, 1]
    zp = xyz[:, :, 2]

    idx3, cx3, cy3, cz3 = _fps_call(xp, yp, zp)
    center_idx = idx3[:, :, 0].T
    cxt = cx3[:, :, 0]
    cyt = cy3[:, :, 0]
    czt = cz3[:, :, 0]
    new_xyz = jnp.stack([cxt.T, cyt.T, czt.T], axis=-1)

    sample_idx = _knn_call(xp, yp, zp, cxt.T, cyt.T, czt.T)

    w1a = W1[:3]
    w1b = W1[3:]
    pq = _pq_call(xyz, features, w1a, w1b, b1.reshape(1, H1))
    table = pq.reshape(B * N, H1)

    flat_idx = (sample_idx
                + (jnp.arange(B, dtype=jnp.int32) * N)[:, None, None])
    flat_idx = flat_idx.reshape(-1)
    g = _gather_call(table, flat_idx)

    g3 = g.reshape(B * NPOINT, NSAMPLE, H1)
    nxf = new_xyz.reshape(B * NPOINT, 3)
    f = _mlp_call(g3, nxf, w1a, W2, b2.reshape(1, H2))
    new_features = f.reshape(B, NPOINT, H2).transpose(0, 2, 1)

    return new_xyz, center_idx, sample_idx, new_features


# speedup vs baseline: 13.9820x; 1.2933x over previous
import functools

import jax
import jax.numpy as jnp
from jax import lax
from jax.experimental import pallas as pl
from jax.experimental.pallas import tpu as pltpu
from jax.experimental.pallas import tpu_sc as plsc

B, N = 4, 8192
NPOINT = 1024
NSAMPLE = 32
H1, H2 = 32, 64

_HI = jax.lax.Precision.HIGHEST


def _fps_body(x_ref, y_ref, z_ref, idx_ref, cx_ref, cy_ref, cz_ref,
              dists_ref, far_ref):
    i = pl.program_id(0)

    @pl.when(i == 0)
    def _init():
        dists_ref[...] = jnp.full((B, N), 1e10, dtype=jnp.float32)
        far_ref[...] = jnp.zeros((B, 1), dtype=jnp.int32)

    far = far_ref[...]
    idx_ref[...] = far.reshape(1, B, 1)

    ji = lax.broadcasted_iota(jnp.int32, (B, N), 1)
    m = ji == far
    X = x_ref[...]
    Y = y_ref[...]
    Z = z_ref[...]
    cx = jnp.sum(jnp.where(m, X, 0.0), axis=1, keepdims=True)
    cy = jnp.sum(jnp.where(m, Y, 0.0), axis=1, keepdims=True)
    cz = jnp.sum(jnp.where(m, Z, 0.0), axis=1, keepdims=True)
    cx_ref[...] = cx.reshape(1, B, 1)
    cy_ref[...] = cy.reshape(1, B, 1)
    cz_ref[...] = cz.reshape(1, B, 1)

    dx = X - cx
    dy = Y - cy
    dz = Z - cz
    d = (dx * dx + dy * dy) + dz * dz
    nd = jnp.minimum(dists_ref[...], d)
    dists_ref[...] = nd
    far_ref[...] = jnp.argmax(nd, axis=1).astype(jnp.int32)[:, None]


def _fps_call(xp, yp, zp):
    plane = pl.BlockSpec((B, N), lambda i: (0, 0))
    out = pl.BlockSpec((1, B, 1), lambda i: (i, 0, 0))
    return pl.pallas_call(
        _fps_body,
        grid=(NPOINT,),
        in_specs=[plane, plane, plane],
        out_specs=[out, out, out, out],
        out_shape=[
            jax.ShapeDtypeStruct((NPOINT, B, 1), jnp.int32),
            jax.ShapeDtypeStruct((NPOINT, B, 1), jnp.float32),
            jax.ShapeDtypeStruct((NPOINT, B, 1), jnp.float32),
            jax.ShapeDtypeStruct((NPOINT, B, 1), jnp.float32),
        ],
        scratch_shapes=[
            pltpu.VMEM((B, N), jnp.float32),
            pltpu.VMEM((B, 1), jnp.int32),
        ],
        compiler_params=pltpu.CompilerParams(
            dimension_semantics=("arbitrary",)),
    )(xp, yp, zp)


_SBLK = 128


def _knn_body(x_ref, y_ref, z_ref, sx_ref, sy_ref, sz_ref, idx_ref):
    X = x_ref[0]
    Y = y_ref[0]
    Z = z_ref[0]
    sx = sx_ref[0]
    sy = sy_ref[0]
    sz = sz_ref[0]
    x2 = (X * X + Y * Y) + Z * Z
    s2 = (sx * sx + sy * sy) + sz * sz
    cen = jnp.concatenate([sx, sy, sz], axis=1)
    xyzt = jnp.concatenate([X, Y, Z], axis=0)
    cross = jax.lax.dot_general(
        cen, xyzt, (((1,), (0,)), ((), ())),
        preferred_element_type=jnp.float32)
    d = (s2 + x2) - 2.0 * cross
    ji = lax.broadcasted_iota(jnp.int32, (_SBLK, N), 1)
    for k in range(NSAMPLE):
        first = jnp.argmin(d, axis=1).astype(jnp.int32)[:, None]
        idx_ref[0, :, k:k + 1] = first
        d = jnp.where(ji == first, jnp.inf, d)


def _knn_call(xp, yp, zp, sxt, syt, szt):
    plane = pl.BlockSpec((1, 1, N), lambda b, s: (b, 0, 0))
    cen = pl.BlockSpec((1, _SBLK, 1), lambda b, s: (b, s, 0))
    p3 = lambda a: a.reshape(B, 1, N)
    c3 = lambda a: a.reshape(B, NPOINT, 1)
    return pl.pallas_call(
        _knn_body,
        grid=(B, NPOINT // _SBLK),
        in_specs=[plane, plane, plane, cen, cen, cen],
        out_specs=pl.BlockSpec((1, _SBLK, NSAMPLE), lambda b, s: (b, s, 0)),
        out_shape=jax.ShapeDtypeStruct((B, NPOINT, NSAMPLE), jnp.int32),
        compiler_params=pltpu.CompilerParams(
            dimension_semantics=("parallel", "arbitrary")),
    )(p3(xp), p3(yp), p3(zp), c3(sxt), c3(syt), c3(szt))


_PBLK = 1024


def _pq_body(xyz_ref, feat_ref, w1a_ref, w1b_ref, b1_ref, pq_ref):
    g = jnp.dot(xyz_ref[0], w1a_ref[...], precision=_HI,
                preferred_element_type=jnp.float32)
    g = g + jnp.dot(feat_ref[0], w1b_ref[...], precision=_HI,
                    preferred_element_type=jnp.float32)
    pq_ref[...] = (g + b1_ref[...]).reshape(1, _PBLK, H1)


def _pq_call(xyz, feat, w1a, w1b, b1r):
    return pl.pallas_call(
        _pq_body,
        grid=(B, N // _PBLK),
        in_specs=[
            pl.BlockSpec((1, _PBLK, 3), lambda b, s: (b, s, 0)),
            pl.BlockSpec((1, _PBLK, feat.shape[-1]), lambda b, s: (b, s, 0)),
            pl.BlockSpec(w1a.shape, lambda b, s: (0, 0)),
            pl.BlockSpec(w1b.shape, lambda b, s: (0, 0)),
            pl.BlockSpec((1, H1), lambda b, s: (0, 0)),
        ],
        out_specs=pl.BlockSpec((1, _PBLK, H1), lambda b, s: (b, s, 0)),
        out_shape=jax.ShapeDtypeStruct((B, N, H1), jnp.float32),
        compiler_params=pltpu.CompilerParams(
            dimension_semantics=("parallel", "parallel")),
    )(xyz, feat, w1a, w1b, b1r)


_GCH = 1024
_NW = 32


def _gather_body(table_ref, idx_ref, out_ref, idx_v, rows_v, sem):
    wid = lax.axis_index("s") * 2 + lax.axis_index("c")
    per_w = (B * NPOINT * NSAMPLE) // _NW
    for ch in range(per_w // _GCH):
        base = wid * per_w + ch * _GCH
        pltpu.sync_copy(idx_ref.at[pl.ds(base, _GCH)], idx_v)
        pltpu.async_copy(table_ref.at[idx_v], rows_v, sem).wait()
        pltpu.sync_copy(rows_v, out_ref.at[pl.ds(base, _GCH)])


def _gather_call(table, flat_idx):
    total = B * NPOINT * NSAMPLE
    mesh = plsc.VectorSubcoreMesh(core_axis_name="c", subcore_axis_name="s")
    return pl.kernel(
        _gather_body,
        out_type=jax.ShapeDtypeStruct((total, H1), jnp.float32),
        mesh=mesh,
        scratch_types=[
            pltpu.VMEM((_GCH,), jnp.int32),
            pltpu.VMEM((_GCH, H1), jnp.float32),
            pltpu.SemaphoreType.DMA,
        ],
        compiler_params=pltpu.CompilerParams(use_tc_tiling_on_sc=False),
    )(table, flat_idx)


_CBLK = 128


def _mlp_body(g_ref, nx_ref, w1a_ref, w2_ref, b2_ref, out_ref):
    r = jnp.dot(nx_ref[...], w1a_ref[...], precision=_HI,
                preferred_element_type=jnp.float32)
    g = g_ref[...]
    h1 = jnp.maximum(g - r[:, None, :], 0.0)
    z = jnp.dot(h1.reshape(_CBLK * NSAMPLE, H1), w2_ref[...], precision=_HI,
                preferred_element_type=jnp.float32)
    z = jnp.maximum(z + b2_ref[...], 0.0)
    zr = z.reshape(_CBLK, NSAMPLE, H2)
    acc = zr[:, 0, :].reshape(_CBLK, H2)
    for j in range(1, NSAMPLE):
        acc = jnp.maximum(acc, zr[:, j, :].reshape(_CBLK, H2))
    out_ref[...] = acc


def _mlp_call(g3, nxf, w1a, w2, b2r):
    s = NPOINT * B // _CBLK
    return pl.pallas_call(
        _mlp_body,
        grid=(s,),
        in_specs=[
            pl.BlockSpec((_CBLK, NSAMPLE, H1), lambda i: (i, 0, 0)),
            pl.BlockSpec((_CBLK, 3), lambda i: (i, 0)),
            pl.BlockSpec(w1a.shape, lambda i: (0, 0)),
            pl.BlockSpec(w2.shape, lambda i: (0, 0)),
            pl.BlockSpec((1, H2), lambda i: (0, 0)),
        ],
        out_specs=pl.BlockSpec((_CBLK, H2), lambda i: (i, 0)),
        out_shape=jax.ShapeDtypeStruct((B * NPOINT, H2), jnp.float32),
        compiler_params=pltpu.CompilerParams(
            dimension_semantics=("arbitrary",)),
    )(g3, nxf, w1a, w2, b2r)


@jax.jit
def kernel(xyz, features, W1, b1, W2, b2):
    xp = xyz[:, :, 0]
    yp = xyz[:, :, 1]
    zp = xyz[:, :, 2]

    idx3, cx3, cy3, cz3 = _fps_call(xp, yp, zp)
    center_idx = idx3[:, :, 0].T
    cxt = cx3[:, :, 0]
    cyt = cy3[:, :, 0]
    czt = cz3[:, :, 0]
    new_xyz = jnp.stack([cxt.T, cyt.T, czt.T], axis=-1)

    sample_idx = _knn_call(xp, yp, zp, cxt.T, cyt.T, czt.T)

    w1a = W1[:3]
    w1b = W1[3:]
    pq = _pq_call(xyz, features, w1a, w1b, b1.reshape(1, H1))
    table = pq.reshape(B * N, H1)

    flat_idx = (sample_idx
                + (jnp.arange(B, dtype=jnp.int32) * N)[:, None, None])
    flat_idx = flat_idx.reshape(-1)
    g = _gather_call(table, flat_idx)

    g3 = g.reshape(B * NPOINT, NSAMPLE, H1)
    nxf = new_xyz.reshape(B * NPOINT, 3)
    f = _mlp_call(g3, nxf, w1a, W2, b2.reshape(1, H2))
    new_features = f.reshape(B, NPOINT, H2).transpose(0, 2, 1)

    return new_xyz, center_idx, sample_idx, new_features


# speedup vs baseline: 14.7555x; 1.0553x over previous
import functools

import jax
import jax.numpy as jnp
from jax import lax
from jax.experimental import pallas as pl
from jax.experimental.pallas import tpu as pltpu
from jax.experimental.pallas import tpu_sc as plsc

B, N = 4, 8192
NPOINT = 1024
NSAMPLE = 32
H1, H2 = 32, 64

_HI = jax.lax.Precision.HIGHEST


_FOLD = 8
_FW = N // _FOLD


def _fps_body(x_ref, y_ref, z_ref, idx_ref, cx_ref, cy_ref, cz_ref,
              dists_ref, far_ref):
    i = pl.program_id(0)

    @pl.when(i == 0)
    def _init():
        dists_ref[...] = jnp.full((B * _FOLD, _FW), 1e10, dtype=jnp.float32)
        far_ref[...] = jnp.zeros((B, 1), dtype=jnp.int32)

    jif = (lax.broadcasted_iota(jnp.int32, (_FOLD, _FW), 0) * _FW
           + lax.broadcasted_iota(jnp.int32, (_FOLD, _FW), 1))
    rows = lax.broadcasted_iota(jnp.int32, (_FOLD, 1), 0)
    idx_parts, cx_parts, cy_parts, cz_parts = [], [], [], []
    for b in range(B):
        fb = far_ref[b:b + 1, 0:1]
        idx_parts.append(fb.reshape(1, 1, 1))
        Xb = x_ref[b * _FOLD:(b + 1) * _FOLD, :]
        Yb = y_ref[b * _FOLD:(b + 1) * _FOLD, :]
        Zb = z_ref[b * _FOLD:(b + 1) * _FOLD, :]
        m = jif == fb
        cx = jnp.sum(jnp.sum(jnp.where(m, Xb, 0.0), axis=1, keepdims=True),
                     axis=0, keepdims=True)
        cy = jnp.sum(jnp.sum(jnp.where(m, Yb, 0.0), axis=1, keepdims=True),
                     axis=0, keepdims=True)
        cz = jnp.sum(jnp.sum(jnp.where(m, Zb, 0.0), axis=1, keepdims=True),
                     axis=0, keepdims=True)
        cx_parts.append(cx.reshape(1, 1, 1))
        cy_parts.append(cy.reshape(1, 1, 1))
        cz_parts.append(cz.reshape(1, 1, 1))
        dx = Xb - cx
        dy = Yb - cy
        dz = Zb - cz
        d = (dx * dx + dy * dy) + dz * dz
        nd = jnp.minimum(dists_ref[b * _FOLD:(b + 1) * _FOLD, :], d)
        dists_ref[b * _FOLD:(b + 1) * _FOLD, :] = nd
        rowmax = jnp.max(nd, axis=1, keepdims=True)
        rowarg = jnp.argmax(nd, axis=1).astype(jnp.int32)[:, None]
        g = jnp.max(rowmax, axis=0, keepdims=True)
        hit = rowmax == g
        rstar = jnp.min(jnp.where(hit, rows, _FOLD), axis=0, keepdims=True)
        arow = jnp.sum(jnp.where(rows == rstar, rowarg, 0), axis=0,
                       keepdims=True)
        far_ref[b:b + 1, 0:1] = rstar * _FW + arow
    idx_ref[...] = jnp.concatenate(idx_parts, axis=1)
    cx_ref[...] = jnp.concatenate(cx_parts, axis=1)
    cy_ref[...] = jnp.concatenate(cy_parts, axis=1)
    cz_ref[...] = jnp.concatenate(cz_parts, axis=1)


def _fps_call(xp, yp, zp):
    plane = pl.BlockSpec((B * _FOLD, _FW), lambda i: (0, 0))
    out = pl.BlockSpec((1, B, 1), lambda i: (i, 0, 0))
    f = lambda a: a.reshape(B * _FOLD, _FW)
    return pl.pallas_call(
        _fps_body,
        grid=(NPOINT,),
        in_specs=[plane, plane, plane],
        out_specs=[out, out, out, out],
        out_shape=[
            jax.ShapeDtypeStruct((NPOINT, B, 1), jnp.int32),
            jax.ShapeDtypeStruct((NPOINT, B, 1), jnp.float32),
            jax.ShapeDtypeStruct((NPOINT, B, 1), jnp.float32),
            jax.ShapeDtypeStruct((NPOINT, B, 1), jnp.float32),
        ],
        scratch_shapes=[
            pltpu.VMEM((B * _FOLD, _FW), jnp.float32),
            pltpu.VMEM((B, 1), jnp.int32),
        ],
        compiler_params=pltpu.CompilerParams(
            dimension_semantics=("arbitrary",)),
    )(f(xp), f(yp), f(zp))


_SBLK = 128


def _knn_body(x_ref, y_ref, z_ref, sx_ref, sy_ref, sz_ref, idx_ref):
    X = x_ref[0]
    Y = y_ref[0]
    Z = z_ref[0]
    sx = sx_ref[0]
    sy = sy_ref[0]
    sz = sz_ref[0]
    x2 = (X * X + Y * Y) + Z * Z
    s2 = (sx * sx + sy * sy) + sz * sz
    cen = jnp.concatenate([sx, sy, sz], axis=1)
    xyzt = jnp.concatenate([X, Y, Z], axis=0)
    cross = jax.lax.dot_general(
        cen, xyzt, (((1,), (0,)), ((), ())),
        preferred_element_type=jnp.float32)
    d = (s2 + x2) - 2.0 * cross
    ji = lax.broadcasted_iota(jnp.int32, (_SBLK, N), 1)
    for k in range(NSAMPLE):
        first = jnp.argmin(d, axis=1).astype(jnp.int32)[:, None]
        idx_ref[0, :, k:k + 1] = first
        d = jnp.where(ji == first, jnp.inf, d)


def _knn_call(xp, yp, zp, sxt, syt, szt):
    plane = pl.BlockSpec((1, 1, N), lambda b, s: (b, 0, 0))
    cen = pl.BlockSpec((1, _SBLK, 1), lambda b, s: (b, s, 0))
    p3 = lambda a: a.reshape(B, 1, N)
    c3 = lambda a: a.reshape(B, NPOINT, 1)
    return pl.pallas_call(
        _knn_body,
        grid=(B, NPOINT // _SBLK),
        in_specs=[plane, plane, plane, cen, cen, cen],
        out_specs=pl.BlockSpec((1, _SBLK, NSAMPLE), lambda b, s: (b, s, 0)),
        out_shape=jax.ShapeDtypeStruct((B, NPOINT, NSAMPLE), jnp.int32),
        compiler_params=pltpu.CompilerParams(
            dimension_semantics=("parallel", "arbitrary")),
    )(p3(xp), p3(yp), p3(zp), c3(sxt), c3(syt), c3(szt))


_PBLK = 1024


def _pq_body(xyz_ref, feat_ref, w1a_ref, w1b_ref, b1_ref, pq_ref):
    g = jnp.dot(xyz_ref[0], w1a_ref[...], precision=_HI,
                preferred_element_type=jnp.float32)
    g = g + jnp.dot(feat_ref[0], w1b_ref[...], precision=_HI,
                    preferred_element_type=jnp.float32)
    pq_ref[...] = (g + b1_ref[...]).reshape(1, _PBLK, H1)


def _pq_call(xyz, feat, w1a, w1b, b1r):
    return pl.pallas_call(
        _pq_body,
        grid=(B, N // _PBLK),
        in_specs=[
            pl.BlockSpec((1, _PBLK, 3), lambda b, s: (b, s, 0)),
            pl.BlockSpec((1, _PBLK, feat.shape[-1]), lambda b, s: (b, s, 0)),
            pl.BlockSpec(w1a.shape, lambda b, s: (0, 0)),
            pl.BlockSpec(w1b.shape, lambda b, s: (0, 0)),
            pl.BlockSpec((1, H1), lambda b, s: (0, 0)),
        ],
        out_specs=pl.BlockSpec((1, _PBLK, H1), lambda b, s: (b, s, 0)),
        out_shape=jax.ShapeDtypeStruct((B, N, H1), jnp.float32),
        compiler_params=pltpu.CompilerParams(
            dimension_semantics=("parallel", "parallel")),
    )(xyz, feat, w1a, w1b, b1r)


_GCH = 1024
_NW = 32


def _gather_body(table_ref, idx_ref, out_ref, idx_v, rows_v, sem):
    wid = lax.axis_index("s") * 2 + lax.axis_index("c")
    per_w = (B * NPOINT * NSAMPLE) // _NW
    for ch in range(per_w // _GCH):
        base = wid * per_w + ch * _GCH
        pltpu.sync_copy(idx_ref.at[pl.ds(base, _GCH)], idx_v)
        pltpu.async_copy(table_ref.at[idx_v], rows_v, sem).wait()
        pltpu.sync_copy(rows_v, out_ref.at[pl.ds(base, _GCH)])


def _gather_call(table, flat_idx):
    total = B * NPOINT * NSAMPLE
    mesh = plsc.VectorSubcoreMesh(core_axis_name="c", subcore_axis_name="s")
    return pl.kernel(
        _gather_body,
        out_type=jax.ShapeDtypeStruct((total, H1), jnp.float32),
        mesh=mesh,
        scratch_types=[
            pltpu.VMEM((_GCH,), jnp.int32),
            pltpu.VMEM((_GCH, H1), jnp.float32),
            pltpu.SemaphoreType.DMA,
        ],
        compiler_params=pltpu.CompilerParams(use_tc_tiling_on_sc=False),
    )(table, flat_idx)


_CBLK = 128


def _mlp_body(g_ref, nx_ref, w1a_ref, w2_ref, b2_ref, out_ref):
    r = jnp.dot(nx_ref[...], w1a_ref[...], precision=_HI,
                preferred_element_type=jnp.float32)
    g = g_ref[...]
    h1 = jnp.maximum(g - r[:, None, :], 0.0)
    z = jnp.dot(h1.reshape(_CBLK * NSAMPLE, H1), w2_ref[...], precision=_HI,
                preferred_element_type=jnp.float32)
    z = jnp.maximum(z + b2_ref[...], 0.0)
    zr = z.reshape(_CBLK, NSAMPLE, H2)
    acc = zr[:, 0, :].reshape(_CBLK, H2)
    for j in range(1, NSAMPLE):
        acc = jnp.maximum(acc, zr[:, j, :].reshape(_CBLK, H2))
    out_ref[...] = acc


def _mlp_call(g3, nxf, w1a, w2, b2r):
    s = NPOINT * B // _CBLK
    return pl.pallas_call(
        _mlp_body,
        grid=(s,),
        in_specs=[
            pl.BlockSpec((_CBLK, NSAMPLE, H1), lambda i: (i, 0, 0)),
            pl.BlockSpec((_CBLK, 3), lambda i: (i, 0)),
            pl.BlockSpec(w1a.shape, lambda i: (0, 0)),
            pl.BlockSpec(w2.shape, lambda i: (0, 0)),
            pl.BlockSpec((1, H2), lambda i: (0, 0)),
        ],
        out_specs=pl.BlockSpec((_CBLK, H2), lambda i: (i, 0)),
        out_shape=jax.ShapeDtypeStruct((B * NPOINT, H2), jnp.float32),
        compiler_params=pltpu.CompilerParams(
            dimension_semantics=("arbitrary",)),
    )(g3, nxf, w1a, w2, b2r)


@jax.jit
def kernel(xyz, features, W1, b1, W2, b2):
    xp = xyz[:, :, 0]
    yp = xyz[:, :, 1]
    zp = xyz[:, :, 2]

    idx3, cx3, cy3, cz3 = _fps_call(xp, yp, zp)
    center_idx = idx3[:, :, 0].T
    cxt = cx3[:, :, 0]
    cyt = cy3[:, :, 0]
    czt = cz3[:, :, 0]
    new_xyz = jnp.stack([cxt.T, cyt.T, czt.T], axis=-1)

    sample_idx = _knn_call(xp, yp, zp, cxt.T, cyt.T, czt.T)

    w1a = W1[:3]
    w1b = W1[3:]
    pq = _pq_call(xyz, features, w1a, w1b, b1.reshape(1, H1))
    table = pq.reshape(B * N, H1)

    flat_idx = (sample_idx
                + (jnp.arange(B, dtype=jnp.int32) * N)[:, None, None])
    flat_idx = flat_idx.reshape(-1)
    g = _gather_call(table, flat_idx)

    g3 = g.reshape(B * NPOINT, NSAMPLE, H1)
    nxf = new_xyz.reshape(B * NPOINT, 3)
    f = _mlp_call(g3, nxf, w1a, W2, b2.reshape(1, H2))
    new_features = f.reshape(B, NPOINT, H2).transpose(0, 2, 1)

    return new_xyz, center_idx, sample_idx, new_features
